# baseline (speedup 1.0000x reference)
import jax
import jax.numpy as jnp
from jax.experimental import pallas as pl

_SA_SPECS = [(1024, 0.1, 32), (256, 0.2, 32), (64, 0.4, 32), (16, 0.8, 32)]


def _mm_kernel(x_ref, w_ref, b_ref, o_ref):
    o_ref[...] = (
        jnp.dot(x_ref[...], w_ref[...], preferred_element_type=jnp.float32)
        + b_ref[...]
    )


def _mm(x, W, b):
    M, K = x.shape
    Co = W.shape[1]
    tm = min(M, 2048)
    return pl.pallas_call(
        _mm_kernel,
        grid=(M // tm,),
        in_specs=[
            pl.BlockSpec((tm, K), lambda i: (i, 0)),
            pl.BlockSpec((K, Co), lambda i: (0, 0)),
            pl.BlockSpec((1, Co), lambda i: (0, 0)),
        ],
        out_specs=pl.BlockSpec((tm, Co), lambda i: (i, 0)),
        out_shape=jax.ShapeDtypeStruct((M, Co), jnp.float32),
    )(x, W, b.reshape(1, -1))


def _head_kernel(h_ref, m_ref, v_ref, g_ref, b_ref, w_ref, b2_ref, o_ref):
    h = h_ref[...]
    x = g_ref[...] * (h - m_ref[...]) / jnp.sqrt(v_ref[...] + 1e-5) + b_ref[...]
    x = jnp.maximum(x, 0.0)
    y = (
        jnp.dot(x, w_ref[...], preferred_element_type=jnp.float32)
        + b2_ref[...]
    )
    z = y - jnp.max(y, axis=1, keepdims=True)
    o_ref[...] = z - jnp.log(jnp.sum(jnp.exp(z), axis=1, keepdims=True))


def _head(h, m, v, g, b, W2, b2):
    M, C = h.shape
    Co = W2.shape[1]
    tm = min(M, 2048)
    row = lambda a: a.reshape(1, -1)
    return pl.pallas_call(
        _head_kernel,
        grid=(M // tm,),
        in_specs=[
            pl.BlockSpec((tm, C), lambda i: (i, 0)),
            pl.BlockSpec((1, C), lambda i: (0, 0)),
            pl.BlockSpec((1, C), lambda i: (0, 0)),
            pl.BlockSpec((1, C), lambda i: (0, 0)),
            pl.BlockSpec((1, C), lambda i: (0, 0)),
            pl.BlockSpec((C, Co), lambda i: (0, 0)),
            pl.BlockSpec((1, Co), lambda i: (0, 0)),
        ],
        out_specs=pl.BlockSpec((tm, Co), lambda i: (i, 0)),
        out_shape=jax.ShapeDtypeStruct((M, Co), jnp.float32),
    )(h, row(m), row(v), row(g), row(b), W2, row(b2))


def _sqd(src, dst):
    return (
        jnp.sum(src**2, -1)[:, :, None]
        + jnp.sum(dst**2, -1)[:, None, :]
        - 2.0 * jnp.einsum("bmc,bnc->bmn", src, dst)
    )


def _fps(xyz, npoint):
    bn, n, _ = xyz.shape

    def body(i, state):
        centroids, distance, farthest = state
        centroids = centroids.at[:, i].set(farthest)
        centroid = jnp.take_along_axis(
            xyz, jnp.broadcast_to(farthest[:, None, None], (bn, 1, 3)), axis=1
        )
        dist = jnp.sum((xyz - centroid) ** 2, -1)
        distance = jnp.minimum(distance, dist)
        farthest = jnp.argmax(distance, axis=-1).astype(jnp.int32)
        return (centroids, distance, farthest)

    state = (
        jnp.zeros((bn, npoint), jnp.int32),
        jnp.full((bn, n), 1e10, jnp.float32),
        jnp.zeros((bn,), jnp.int32),
    )
    centroids, _, _ = jax.lax.fori_loop(0, npoint, body, state)
    return centroids


def _ball_query(new_xyz, xyz, radius, nsample):
    bn, s, _ = new_xyz.shape
    n = xyz.shape[1]
    sqrdists = _sqd(new_xyz, xyz)
    gi = jnp.broadcast_to(jnp.arange(n, dtype=jnp.int32), (bn, s, n))
    gi = jnp.where(sqrdists > radius**2, n, gi)
    gi = jnp.sort(gi, axis=-1)[:, :, :nsample]
    first = gi[:, :, :1]
    return jnp.where(gi == n, first, gi)


def _bn_jax(x, g, b, axes):
    m = jnp.mean(x, axis=axes, keepdims=True)
    v = jnp.mean((x - m) ** 2, axis=axes, keepdims=True)
    return g * (x - m) / jnp.sqrt(v + 1e-5) + b


def _gather(points, idx):
    return jax.vmap(lambda p, i: p[i])(points, idx)


def _sa_stage(layers, npoint, radius, nsample, xyz, points):
    fps_idx = _fps(xyz, npoint)
    new_xyz = _gather(xyz, fps_idx)
    idx = _ball_query(new_xyz, xyz, radius, nsample)
    g_xyz = _gather(xyz, idx) - new_xyz[:, :, None, :]
    g_pts = _gather(points, idx)
    h = jnp.concatenate([g_xyz, g_pts], axis=-1)
    for (W, bb, g, bt) in layers:
        h = jnp.einsum("bskc,co->bsko", h, W) + bb
        h = jax.nn.relu(_bn_jax(h, g, bt, (0, 1, 2)))
    return new_xyz, jnp.max(h, axis=2)


def _fp_stage(layers, xyz1, xyz2, points1, points2):
    dists = _sqd(xyz1, xyz2)
    idx = jnp.argsort(dists, axis=-1)[:, :, :3]
    d = jnp.take_along_axis(dists, idx, axis=-1)
    recip = 1.0 / (d + 1e-8)
    w = recip / jnp.sum(recip, axis=2, keepdims=True)
    interp = jnp.sum(_gather(points2, idx) * w[..., None], axis=2)
    h = interp if points1 is None else jnp.concatenate([points1, interp], axis=-1)
    for (W, bb, g, bt) in layers:
        h = jnp.einsum("bnc,co->bno", h, W) + bb
        h = jax.nn.relu(_bn_jax(h, g, bt, (0, 1)))
    return h


def _forward_impl(xyz, params):
    pts = jnp.transpose(xyz, (0, 2, 1))
    l0_xyz = pts[:, :, :3]
    l0_points = pts
    l1_xyz, l1_points = _sa_stage(params["sa1"], *_SA_SPECS[0], l0_xyz, l0_points)
    l2_xyz, l2_points = _sa_stage(params["sa2"], *_SA_SPECS[1], l1_xyz, l1_points)
    l3_xyz, l3_points = _sa_stage(params["sa3"], *_SA_SPECS[2], l2_xyz, l2_points)
    l4_xyz, l4_points = _sa_stage(params["sa4"], *_SA_SPECS[3], l3_xyz, l3_points)
    l3p = _fp_stage(params["fp4"], l3_xyz, l4_xyz, l3_points, l4_points)
    l2p = _fp_stage(params["fp3"], l2_xyz, l3_xyz, l2_points, l3p)
    l1p = _fp_stage(params["fp2"], l1_xyz, l2_xyz, l1_points, l2p)
    l0p = _fp_stage(params["fp1"], l0_xyz, l1_xyz, None, l1p)
    W1, b1 = params["conv1"]
    g1, bt1 = params["bn1"]
    bn, n, _ = l0p.shape
    h = _mm(l0p.reshape(bn * n, -1), W1, b1)
    hr = h.reshape(bn, n, -1)
    m = jnp.mean(hr, axis=(0, 1))
    v = jnp.mean((hr - m[None, None, :]) ** 2, axis=(0, 1))
    W2, b2 = params["conv2"]
    out = _head(h, m, v, g1, bt1, W2, b2).reshape(bn, n, -1)
    return out, jnp.transpose(l4_points, (0, 2, 1))


def kernel(xyz, params):
    return jax.jit(_forward_impl)(xyz, params)
